# baseline (speedup 1.0000x reference)
import functools

import jax
import jax.numpy as jnp
from jax import lax
from jax.experimental import pallas as pl
from jax.experimental.pallas import tpu as pltpu
from jax.experimental.pallas import tpu_sc as plsc

NUM_ENTITIES = 1000000
NUM_RELATIONS = 1000
DIM = 64
B = 16384

_info = plsc.get_sparse_core_info()
_NC = _info.num_cores
_NS = _info.num_subcores
_NW = _NC * _NS
_BPW = B // _NW
_CH = 128
_NCHUNK = _BPW // _CH


def _build():
  mesh = plsc.VectorSubcoreMesh(core_axis_name="c", subcore_axis_name="s")
  out_t = jax.ShapeDtypeStruct((B, DIM), jnp.float32)

  @functools.partial(
      pl.kernel,
      mesh=mesh,
      compiler_params=pltpu.CompilerParams(use_tc_tiling_on_sc=False),
      out_type=(out_t, out_t, out_t),
      scratch_types=[
          pltpu.VMEM((_NCHUNK, _CH), jnp.int32),
          pltpu.VMEM((_NCHUNK, _CH), jnp.int32),
          pltpu.VMEM((_NCHUNK, _CH), jnp.int32),
          pltpu.VMEM((_BPW, DIM), jnp.float32),
          pltpu.VMEM((_BPW, DIM), jnp.float32),
          pltpu.VMEM((_BPW, DIM), jnp.float32),
          pltpu.SemaphoreType.DMA,
          pltpu.SemaphoreType.DMA,
      ],
  )
  def gather3(heads_hbm, rels_hbm, tails_hbm, ent_hbm, rel_hbm,
              out_h, out_r, out_t_ref,
              idx_h, idx_r, idx_t, rows_h, rows_r, rows_t, gsem, ssem):
    wid = lax.axis_index("s") * _NC + lax.axis_index("c")
    base = wid * _BPW

    pltpu.sync_copy(heads_hbm.at[wid], idx_h)
    pltpu.sync_copy(rels_hbm.at[wid], idx_r)
    pltpu.sync_copy(tails_hbm.at[wid], idx_t)

    copies = []
    for j in range(_NCHUNK):
      copies.append(pltpu.async_copy(
          ent_hbm.at[idx_h.at[j]], rows_h.at[pl.ds(j * _CH, _CH)], gsem))
      copies.append(pltpu.async_copy(
          rel_hbm.at[idx_r.at[j]], rows_r.at[pl.ds(j * _CH, _CH)], gsem))
      copies.append(pltpu.async_copy(
          ent_hbm.at[idx_t.at[j]], rows_t.at[pl.ds(j * _CH, _CH)], gsem))
    for c in copies:
      c.wait()

    st = [
        pltpu.async_copy(rows_h, out_h.at[pl.ds(base, _BPW)], ssem),
        pltpu.async_copy(rows_r, out_r.at[pl.ds(base, _BPW)], ssem),
        pltpu.async_copy(rows_t, out_t_ref.at[pl.ds(base, _BPW)], ssem),
    ]
    for c in st:
      c.wait()

  return gather3


_GATHER3 = _build()


def kernel(heads, relations, tails, entity_embeddings, relation_embeddings):
  h = heads.astype(jnp.int32).reshape(_NW, _NCHUNK, _CH)
  r = relations.astype(jnp.int32).reshape(_NW, _NCHUNK, _CH)
  t = tails.astype(jnp.int32).reshape(_NW, _NCHUNK, _CH)
  return _GATHER3(h, r, t, entity_embeddings, relation_embeddings)


# speedup vs baseline: 1.3235x; 1.3235x over previous
import functools

import jax
import jax.numpy as jnp
from jax import lax
from jax.experimental import pallas as pl
from jax.experimental.pallas import tpu as pltpu
from jax.experimental.pallas import tpu_sc as plsc

NUM_ENTITIES = 1000000
NUM_RELATIONS = 1000
DIM = 64
B = 16384
SUB = 8

_info = plsc.get_sparse_core_info()
_NC = _info.num_cores
_NS = _info.num_subcores
_NW = _NC * _NS
_BPW = B // _NW
_G = 16
_NG = _BPW // _G


def _build():
  mesh = plsc.VectorSubcoreMesh(core_axis_name="c", subcore_axis_name="s")
  out_t = jax.ShapeDtypeStruct((B, DIM), jnp.float32)

  @functools.partial(
      pl.kernel,
      mesh=mesh,
      compiler_params=pltpu.CompilerParams(
          use_tc_tiling_on_sc=True, needs_layout_passes=False),
      out_type=(out_t, out_t, out_t),
      scratch_types=[
          pltpu.VMEM((_BPW,), jnp.int32),
          pltpu.SMEM((_BPW,), jnp.int32),
          pltpu.VMEM((_G, SUB, DIM), jnp.float32),
          pltpu.VMEM((_G, SUB, DIM), jnp.float32),
          pltpu.VMEM((_G, DIM), jnp.float32),
          pltpu.VMEM((_G, DIM), jnp.float32),
          pltpu.SemaphoreType.DMA,
          pltpu.SemaphoreType.DMA,
          pltpu.SemaphoreType.DMA,
          pltpu.SemaphoreType.DMA,
      ],
  )
  def gather3(heads_hbm, rels_hbm, tails_hbm, ent_hbm, rel_hbm,
              out_h, out_r, out_tl,
              idx_v, idx_s, gb0, gb1, rb0, rb1, gsem0, gsem1, ssem0, ssem1):
    wid = lax.axis_index("s") * _NC + lax.axis_index("c")
    base = wid * _BPW
    gbufs = (gb0, gb1)
    rbufs = (rb0, rb1)
    gsems = (gsem0, gsem1)
    ssems = (ssem0, ssem1)

    for idx_hbm, tbl, out in ((heads_hbm, ent_hbm, out_h),
                              (rels_hbm, rel_hbm, out_r),
                              (tails_hbm, ent_hbm, out_tl)):
      pltpu.sync_copy(idx_hbm.at[pl.ds(base, _BPW)], idx_v)

      lanes = lax.iota(jnp.int32, 16)

      def scalarize(v):
        vec = idx_v[pl.ds(v * 16, 16)]
        for k in range(16):
          s = lax.reduce_sum_p.bind(
              jnp.where(lanes == k, vec, 0), axes=(0,))
          idx_s[v * 16 + k] = s

      pl.loop(0, _BPW // 16)(scalarize)

      def issue(g, p):
        for k in range(_G):
          i = idx_s[g * _G + k]
          t8 = pl.multiple_of((i >> 3) * SUB, SUB)
          pltpu.async_copy(tbl.at[pl.ds(t8, SUB)], gbufs[p].at[k], gsems[p])

      def wait_store(p):
        pltpu.make_async_copy(
            rbufs[p], out.at[pl.ds(base, _G)], ssems[p]).wait()

      def consume(g, p):
        for k in range(_G):
          pltpu.make_async_copy(
              tbl.at[pl.ds(0, SUB)], gbufs[p].at[k], gsems[p]).wait()
        for k in range(_G):
          j = idx_s[g * _G + k] & 7
          for q in range(DIM // 16):
            sq = pl.ds(16 * q, 16)
            rbufs[p][k, sq] = gbufs[p][k, j, sq]
        dst = pl.multiple_of(base + g * _G, SUB)
        pltpu.async_copy(rbufs[p], out.at[pl.ds(dst, _G)], ssems[p])

      issue(0, 0)

      def pair(h):
        g0 = 2 * h
        g1 = g0 + 1
        issue(g1, 1)
        @pl.when(h > 0)
        def _():
          wait_store(0)
        consume(g0, 0)
        @pl.when(g1 + 1 < _NG)
        def _():
          issue(g1 + 1, 0)
        @pl.when(h > 0)
        def _():
          wait_store(1)
        consume(g1, 1)

      pl.loop(0, _NG // 2)(pair)
      wait_store(0)
      wait_store(1)

  return gather3


_GATHER3 = _build()


def kernel(heads, relations, tails, entity_embeddings, relation_embeddings):
  h = heads.astype(jnp.int32)
  r = relations.astype(jnp.int32)
  t = tails.astype(jnp.int32)
  return _GATHER3(h, r, t, entity_embeddings, relation_embeddings)


# speedup vs baseline: 1.7609x; 1.3304x over previous
import functools

import jax
import jax.numpy as jnp
from jax import lax
from jax.experimental import pallas as pl
from jax.experimental.pallas import tpu as pltpu
from jax.experimental.pallas import tpu_sc as plsc

NUM_ENTITIES = 1000000
NUM_RELATIONS = 1000
DIM = 64
B = 16384
SUB = 8

_info = plsc.get_sparse_core_info()
_NC = _info.num_cores
_NS = _info.num_subcores
_NW = _NC * _NS
_BPW = B // _NW
_G = 16
_NG = _BPW // _G


def _build():
  mesh = plsc.VectorSubcoreMesh(core_axis_name="c", subcore_axis_name="s")
  out_t = jax.ShapeDtypeStruct((B, DIM), jnp.float32)

  @functools.partial(
      pl.kernel,
      mesh=mesh,
      compiler_params=pltpu.CompilerParams(
          use_tc_tiling_on_sc=True, needs_layout_passes=False),
      out_type=(out_t, out_t, out_t),
      scratch_types=[
          pltpu.VMEM((_BPW,), jnp.int32),
          pltpu.SMEM((_BPW,), jnp.int32),
          pltpu.VMEM((_G, SUB, DIM), jnp.float32),
          pltpu.VMEM((_G, SUB, DIM), jnp.float32),
          pltpu.VMEM((_G, DIM), jnp.float32),
          pltpu.VMEM((_G, DIM), jnp.float32),
          pltpu.SemaphoreType.DMA,
          pltpu.SemaphoreType.DMA,
          pltpu.SemaphoreType.DMA,
          pltpu.SemaphoreType.DMA,
      ],
  )
  def gather3(heads_hbm, rels_hbm, tails_hbm, ent_hbm, rel_hbm,
              out_h, out_r, out_tl,
              idx_v, idx_s, gb0, gb1, rb0, rb1, gsem0, gsem1, ssem0, ssem1):
    wid = lax.axis_index("s") * _NC + lax.axis_index("c")
    base = wid * _BPW
    gbufs = (gb0, gb1)
    rbufs = (rb0, rb1)
    gsems = (gsem0, gsem1)
    ssems = (ssem0, ssem1)

    for idx_hbm, tbl, out in ((heads_hbm, ent_hbm, out_h),
                              (rels_hbm, rel_hbm, out_r),
                              (tails_hbm, ent_hbm, out_tl)):
      pltpu.sync_copy(idx_hbm.at[pl.ds(base, _BPW)], idx_v)

      lanes = lax.iota(jnp.int32, 16)

      def scalarize(v):
        vec = idx_v[pl.ds(v * 16, 16)]
        for k in range(16):
          s = lax.reduce_sum_p.bind(
              jnp.where(lanes == k, vec, 0), axes=(0,))
          idx_s[v * 16 + k] = s

      pl.loop(0, _BPW // 16)(scalarize)

      def issue(g, p):
        for k in range(_G):
          i = idx_s[g * _G + k]
          pltpu.async_copy(tbl.at[i >> 3], gbufs[p].at[k], gsems[p])

      def wait_store(p):
        pltpu.make_async_copy(
            rbufs[p], out.at[pl.ds(base, _G)], ssems[p]).wait()

      def consume(g, p):
        for k in range(_G):
          pltpu.make_async_copy(
              tbl.at[0], gbufs[p].at[k], gsems[p]).wait()
        for k in range(_G):
          j = idx_s[g * _G + k] & 7
          for q in range(DIM // 16):
            sq = pl.ds(16 * q, 16)
            rbufs[p][k, sq] = gbufs[p][k, j, sq]
        dst = pl.multiple_of(base + g * _G, SUB)
        pltpu.async_copy(rbufs[p], out.at[pl.ds(dst, _G)], ssems[p])

      issue(0, 0)

      def pair(h):
        g0 = 2 * h
        g1 = g0 + 1
        issue(g1, 1)
        @pl.when(h > 0)
        def _():
          wait_store(0)
        consume(g0, 0)
        @pl.when(g1 + 1 < _NG)
        def _():
          issue(g1 + 1, 0)
        @pl.when(h > 0)
        def _():
          wait_store(1)
        consume(g1, 1)

      pl.loop(0, _NG // 2)(pair)
      wait_store(0)
      wait_store(1)

  return gather3


_GATHER3 = _build()


def kernel(heads, relations, tails, entity_embeddings, relation_embeddings):
  ent3 = entity_embeddings.reshape(NUM_ENTITIES // SUB, SUB, DIM)
  rel3 = relation_embeddings.reshape(NUM_RELATIONS // SUB, SUB, DIM)
  h = heads.astype(jnp.int32)
  r = relations.astype(jnp.int32)
  t = tails.astype(jnp.int32)
  return _GATHER3(h, r, t, ent3, rel3)


# speedup vs baseline: 1.7668x; 1.0033x over previous
import functools

import jax
import jax.numpy as jnp
from jax import lax
from jax.experimental import pallas as pl
from jax.experimental.pallas import tpu as pltpu
from jax.experimental.pallas import tpu_sc as plsc

NUM_ENTITIES = 1000000
NUM_RELATIONS = 1000
DIM = 64
B = 16384
SUB = 8

_info = plsc.get_sparse_core_info()
_NC = _info.num_cores
_NS = _info.num_subcores
_NW = _NC * _NS
_BPW = B // _NW
_G = 16
_NG = _BPW // _G


def _build():
  mesh = plsc.VectorSubcoreMesh(core_axis_name="c", subcore_axis_name="s")
  out_t = jax.ShapeDtypeStruct((B, DIM), jnp.float32)

  @functools.partial(
      pl.kernel,
      mesh=mesh,
      compiler_params=pltpu.CompilerParams(
          use_tc_tiling_on_sc=True, needs_layout_passes=False),
      out_type=(out_t, out_t, out_t),
      scratch_types=[
          pltpu.VMEM((_BPW,), jnp.int32),
          pltpu.VMEM((_G, SUB, DIM), jnp.float32),
          pltpu.VMEM((_G, SUB, DIM), jnp.float32),
          pltpu.VMEM((_G, DIM), jnp.float32),
          pltpu.VMEM((_G, DIM), jnp.float32),
          pltpu.SemaphoreType.DMA,
          pltpu.SemaphoreType.DMA,
          pltpu.SemaphoreType.DMA,
          pltpu.SemaphoreType.DMA,
      ],
  )
  def gather3(heads_hbm, rels_hbm, tails_hbm, ent_hbm, rel_hbm,
              out_h, out_r, out_tl,
              idx_v, gb0, gb1, rb0, rb1, gsem0, gsem1, ssem0, ssem1):
    wid = lax.axis_index("s") * _NC + lax.axis_index("c")
    base = wid * _BPW
    gbufs = (gb0, gb1)
    rbufs = (rb0, rb1)
    gsems = (gsem0, gsem1)
    ssems = (ssem0, ssem1)

    for idx_hbm, tbl, out in ((heads_hbm, ent_hbm, out_h),
                              (rels_hbm, rel_hbm, out_r),
                              (tails_hbm, ent_hbm, out_tl)):
      pltpu.sync_copy(idx_hbm.at[pl.ds(base, _BPW)], idx_v)

      def issue(g, p):
        tvec = lax.shift_right_logical(idx_v[pl.ds(g * _G, _G)], 3)
        for k in range(_G):
          pltpu.async_copy(tbl.at[tvec[k]], gbufs[p].at[k], gsems[p])

      def wait_store(p):
        pltpu.make_async_copy(
            rbufs[p], out.at[pl.ds(base, _G)], ssems[p]).wait()

      def consume(g, p):
        for k in range(_G):
          pltpu.make_async_copy(
              tbl.at[0], gbufs[p].at[k], gsems[p]).wait()
        jvec = idx_v[pl.ds(g * _G, _G)] & 7
        for k in range(_G):
          j = jvec[k]
          for q in range(DIM // 16):
            sq = pl.ds(16 * q, 16)
            rbufs[p][k, sq] = gbufs[p][k, j, sq]
        dst = pl.multiple_of(base + g * _G, SUB)
        pltpu.async_copy(rbufs[p], out.at[pl.ds(dst, _G)], ssems[p])

      issue(0, 0)

      def pair(h):
        g0 = 2 * h
        g1 = g0 + 1
        issue(g1, 1)
        @pl.when(h > 0)
        def _():
          wait_store(0)
        consume(g0, 0)
        @pl.when(g1 + 1 < _NG)
        def _():
          issue(g1 + 1, 0)
        @pl.when(h > 0)
        def _():
          wait_store(1)
        consume(g1, 1)

      pl.loop(0, _NG // 2)(pair)
      wait_store(0)
      wait_store(1)

  return gather3


_GATHER3 = _build()


def kernel(heads, relations, tails, entity_embeddings, relation_embeddings):
  ent3 = entity_embeddings.reshape(NUM_ENTITIES // SUB, SUB, DIM)
  rel3 = relation_embeddings.reshape(NUM_RELATIONS // SUB, SUB, DIM)
  h = heads.astype(jnp.int32)
  r = relations.astype(jnp.int32)
  t = tails.astype(jnp.int32)
  return _GATHER3(h, r, t, ent3, rel3)


# speedup vs baseline: 1.7858x; 1.0108x over previous
import functools

import jax
import jax.numpy as jnp
from jax import lax
from jax.experimental import pallas as pl
from jax.experimental.pallas import tpu as pltpu
from jax.experimental.pallas import tpu_sc as plsc

NUM_ENTITIES = 1000000
NUM_RELATIONS = 1000
DIM = 64
B = 16384
SUB = 8

_info = plsc.get_sparse_core_info()
_NC = _info.num_cores
_NS = _info.num_subcores
_NW = _NC * _NS
_BPW = B // _NW
_G = 32
_NG = _BPW // _G


def _build():
  mesh = plsc.VectorSubcoreMesh(core_axis_name="c", subcore_axis_name="s")
  out_t = jax.ShapeDtypeStruct((B, DIM), jnp.float32)

  @functools.partial(
      pl.kernel,
      mesh=mesh,
      compiler_params=pltpu.CompilerParams(
          use_tc_tiling_on_sc=True, needs_layout_passes=False),
      out_type=(out_t, out_t, out_t),
      scratch_types=[
          pltpu.VMEM((_BPW,), jnp.int32),
          pltpu.VMEM((_G, SUB, DIM), jnp.float32),
          pltpu.VMEM((_G, SUB, DIM), jnp.float32),
          pltpu.VMEM((_G, DIM), jnp.float32),
          pltpu.VMEM((_G, DIM), jnp.float32),
          pltpu.SemaphoreType.DMA,
          pltpu.SemaphoreType.DMA,
          pltpu.SemaphoreType.DMA,
          pltpu.SemaphoreType.DMA,
      ],
  )
  def gather3(heads_hbm, rels_hbm, tails_hbm, ent_hbm, rel_hbm,
              out_h, out_r, out_tl,
              idx_v, gb0, gb1, rb0, rb1, gsem0, gsem1, ssem0, ssem1):
    wid = lax.axis_index("s") * _NC + lax.axis_index("c")
    base = wid * _BPW
    gbufs = (gb0, gb1)
    rbufs = (rb0, rb1)
    gsems = (gsem0, gsem1)
    ssems = (ssem0, ssem1)

    for idx_hbm, tbl, out in ((heads_hbm, ent_hbm, out_h),
                              (rels_hbm, rel_hbm, out_r),
                              (tails_hbm, ent_hbm, out_tl)):
      pltpu.sync_copy(idx_hbm.at[pl.ds(base, _BPW)], idx_v)

      def issue(g, p):
        for v in range(_G // 16):
          tvec = lax.shift_right_logical(
              idx_v[pl.ds(g * _G + 16 * v, 16)], 3)
          for k in range(16):
            pltpu.async_copy(
                tbl.at[tvec[k]], gbufs[p].at[16 * v + k], gsems[p])

      def wait_store(p):
        pltpu.make_async_copy(
            rbufs[p], out.at[pl.ds(base, _G)], ssems[p]).wait()

      def consume(g, p):
        for k in range(_G):
          pltpu.make_async_copy(
              tbl.at[0], gbufs[p].at[k], gsems[p]).wait()
        for v in range(_G // 16):
          jvec = idx_v[pl.ds(g * _G + 16 * v, 16)] & 7
          for k in range(16):
            j = jvec[k]
            for q in range(DIM // 16):
              sq = pl.ds(16 * q, 16)
              rbufs[p][16 * v + k, sq] = gbufs[p][16 * v + k, j, sq]
        dst = pl.multiple_of(base + g * _G, SUB)
        pltpu.async_copy(rbufs[p], out.at[pl.ds(dst, _G)], ssems[p])

      issue(0, 0)

      def pair(h):
        g0 = 2 * h
        g1 = g0 + 1
        issue(g1, 1)
        @pl.when(h > 0)
        def _():
          wait_store(0)
        consume(g0, 0)
        @pl.when(g1 + 1 < _NG)
        def _():
          issue(g1 + 1, 0)
        @pl.when(h > 0)
        def _():
          wait_store(1)
        consume(g1, 1)

      pl.loop(0, _NG // 2)(pair)
      wait_store(0)
      wait_store(1)

  return gather3


_GATHER3 = _build()


def kernel(heads, relations, tails, entity_embeddings, relation_embeddings):
  ent3 = entity_embeddings.reshape(NUM_ENTITIES // SUB, SUB, DIM)
  rel3 = relation_embeddings.reshape(NUM_RELATIONS // SUB, SUB, DIM)
  h = heads.astype(jnp.int32)
  r = relations.astype(jnp.int32)
  t = tails.astype(jnp.int32)
  return _GATHER3(h, r, t, ent3, rel3)
